# Initial kernel scaffold; baseline (speedup 1.0000x reference)
import jax
import jax.numpy as jnp
from jax.experimental import pallas as pl


def kernel(obj_logits, rel_pair_idxs, obj_embed, kg_keys, kg_vals):
    raise NotImplementedError("write your pallas kernel here")


# speedup vs baseline: 7.4511x; 7.4511x over previous
import functools

import jax
import jax.numpy as jnp
from jax import lax
from jax.experimental import pallas as pl
from jax.experimental.pallas import tpu as pltpu
from jax.experimental.pallas import tpu_sc as plsc

N_OBJ = 1024
NUM_OBJ = 151
NUM_OBJ_PAD = 256
NUM_REL = 51
NUM_REL_PAD = 64
VAL_PAD = 128
D = 128
KG = 100000
BLK = 128
NB = 784
KG_PAD = NB * BLK
NB_PAD = 896
KB = 512
GSUB = KB // BLK
NSTEP = KG_PAD // KB
TOPK = 32
NEG = -1e30


def _prelude_body(l_ref, e_ref, i_ref, q_ref):
    lv = l_ref[...]
    m = jnp.max(lv, axis=1, keepdims=True)
    ex = jnp.exp(lv - m)
    p = ex / jnp.sum(ex, axis=1, keepdims=True)
    f = jnp.dot(p, e_ref[...], preferred_element_type=jnp.float32)
    ii = i_ref[...]
    col = lax.broadcasted_iota(jnp.int32, (N_OBJ, N_OBJ), 1)
    oh0 = jnp.where(col == ii[:, 0:1], 1.0, 0.0)
    oh1 = jnp.where(col == ii[:, 1:2], 1.0, 0.0)
    h = jnp.dot(oh0, f, preferred_element_type=jnp.float32,
                precision=lax.Precision.HIGHEST)
    t = jnp.dot(oh1, f, preferred_element_type=jnp.float32,
                precision=lax.Precision.HIGHEST)
    q = h * t
    nrm = jnp.sqrt(jnp.sum(q * q, axis=1, keepdims=True))
    q_ref[...] = q / (nrm + 1e-6)


def _prelude(logits_pad, embed_pad, idx):
    return pl.pallas_call(
        _prelude_body,
        out_shape=jax.ShapeDtypeStruct((N_OBJ, D), jnp.float32),
    )(logits_pad, embed_pad, idx)


def _sims_body(q_ref, k_ref, s_ref, m_ref):
    j = pl.program_id(0)
    s = lax.dot_general(q_ref[...], k_ref[...], (((1,), (1,)), ((), ())),
                        preferred_element_type=jnp.float32)
    cg = j * KB + lax.broadcasted_iota(jnp.int32, (1, KB), 1)
    s = jnp.where(cg < KG, s, NEG)
    s_ref[...] = s
    parts = [jnp.max(s[:, g * BLK:(g + 1) * BLK], axis=1, keepdims=True)
             for g in range(GSUB)]
    m_ref[0] = jnp.concatenate(parts, axis=1)


def _sims_and_blockmax(qn, keys_pad):
    return pl.pallas_call(
        _sims_body,
        grid=(NSTEP,),
        in_specs=[
            pl.BlockSpec((N_OBJ, D), lambda j: (0, 0)),
            pl.BlockSpec((KB, D), lambda j: (j, 0)),
        ],
        out_specs=[
            pl.BlockSpec((N_OBJ, KB), lambda j: (0, j)),
            pl.BlockSpec((1, N_OBJ, GSUB), lambda j: (j, 0, 0)),
        ],
        out_shape=[
            jax.ShapeDtypeStruct((N_OBJ, KG_PAD), jnp.float32),
            jax.ShapeDtypeStruct((NSTEP, N_OBJ, GSUB), jnp.float32),
        ],
    )(qn, keys_pad)


def _select_body(m_ref, bi_ref, bv_ref):
    a0 = m_ref[...]
    col = lax.broadcasted_iota(jnp.int32, (N_OBJ, NB_PAD), 1)
    k32 = lax.broadcasted_iota(jnp.int32, (N_OBJ, TOPK), 1)

    def body(k, carry):
        a, bi, bv = carry
        m = jnp.max(a, axis=1)
        am = jnp.min(jnp.where(a == m[:, None], col, 2**30), axis=1)
        bi = jnp.where(k32 == k, am[:, None], bi)
        bv = jnp.where(k32 == k, m[:, None], bv)
        a = jnp.where(col == am[:, None], -3e38, a)
        return (a, bi, bv)

    _, bi, bv = lax.fori_loop(
        0, TOPK, body,
        (a0, jnp.zeros((N_OBJ, TOPK), jnp.int32),
         jnp.zeros((N_OBJ, TOPK), jnp.float32)))
    bi_ref[...] = bi
    bv_ref[...] = bv


def _select_blocks(m_pad):
    return pl.pallas_call(
        _select_body,
        out_shape=[
            jax.ShapeDtypeStruct((N_OBJ, TOPK), jnp.int32),
            jax.ShapeDtypeStruct((N_OBJ, TOPK), jnp.float32),
        ],
    )(m_pad)


ROWS_PER_W = 32


def _splat_i(x):
    return jnp.full((16,), x, jnp.int32)


def _sc_body(sims_ref, bi_ref, bv_ref, vals_ref, out_ref,
             blk_v, sel_v, gid_v, simsb, selrow, topv, topi, valsv,
             outrow, sem):
    wid = lax.axis_index("s") * 2 + lax.axis_index("c")
    base = wid * ROWS_PER_W
    pltpu.sync_copy(bi_ref.at[pl.ds(base, ROWS_PER_W)], blk_v)
    pltpu.sync_copy(bv_ref.at[pl.ds(base, ROWS_PER_W)], sel_v)
    lane = lax.broadcasted_iota(jnp.int32, (16,), 0)

    def row_body(r, _):
        row = base + r
        rsp = _splat_i(r)
        for c in range(2):
            b = plsc.load_gather(blk_v, [rsp, lane + (c * 16)])
            gid_v[pl.ds(c * 16, 16)] = b + row * NB
            selrow[pl.ds(c * 16, 16)] = plsc.load_gather(
                sel_v, [rsp, lane + (c * 16)])
        pltpu.async_copy(sims_ref.at[gid_v], simsb, sem).wait()

        def sel_body(k, _):
            b0 = selrow[pl.ds(0, 16)]
            b1 = selrow[pl.ds(16, 16)]
            m = jnp.max(jnp.maximum(b0, b1))
            c0 = jnp.min(plsc.all_reduce_ffs(b0 == m))
            c1 = jnp.min(plsc.all_reduce_ffs(b1 == m))
            sstar = jnp.where(c0 < 16, c0, c1 + 16)
            ssp = _splat_i(sstar)
            pos = jnp.int32(1 << 20)
            for c in range(8):
                v = plsc.load_gather(simsb, [ssp, lane + (c * 16)])
                f = jnp.min(plsc.all_reduce_ffs(v == m))
                pos = jnp.minimum(pos,
                                  jnp.where(f < 16, c * 16 + f, 1 << 20))
            mask0 = lane == 0
            plsc.store_scatter(topv, [_splat_i(k)],
                               jnp.full((16,), m, jnp.float32), mask=mask0)
            blkid = jnp.max(plsc.load_gather(blk_v, [rsp, ssp]))
            plsc.store_scatter(topi, [_splat_i(k)],
                               _splat_i(blkid * BLK + pos), mask=mask0)
            plsc.store_scatter(simsb, [ssp, _splat_i(pos)],
                               jnp.full((16,), NEG, jnp.float32), mask=mask0)
            nm = jnp.full((16,), -3e38, jnp.float32)
            for c in range(8):
                nm = jnp.maximum(
                    nm, plsc.load_gather(simsb, [ssp, lane + (c * 16)]))
            plsc.store_scatter(selrow, [ssp],
                               jnp.full((16,), jnp.max(nm), jnp.float32),
                               mask=mask0)
            return 0

        lax.fori_loop(0, TOPK, sel_body, 0)

        t0 = topv[pl.ds(0, 16)]
        t1 = topv[pl.ds(16, 16)]
        m1 = jnp.max(jnp.maximum(t0, t1))
        w0 = jnp.exp((t0 - m1) * 10.0)
        w1 = jnp.exp((t1 - m1) * 10.0)
        ssum = jnp.full((16,), jnp.sum(w0) + jnp.sum(w1), jnp.float32)
        topv[pl.ds(0, 16)] = w0 / ssum
        topv[pl.ds(16, 16)] = w1 / ssum

        pltpu.async_copy(vals_ref.at[topi], valsv, sem).wait()
        for c in range(4):
            outrow[pl.ds(c * 16, 16)] = jnp.zeros((16,), jnp.float32)

        def acc_body(k, _):
            ksp = _splat_i(k)
            wk = jnp.max(plsc.load_gather(topv, [ksp]))
            for c in range(4):
                v = plsc.load_gather(valsv, [ksp, lane + (c * 16)])
                outrow[pl.ds(c * 16, 16)] = outrow[pl.ds(c * 16, 16)] + v * wk
            return 0

        lax.fori_loop(0, TOPK, acc_body, 0)
        pltpu.sync_copy(outrow, out_ref.at[row])
        return 0

    lax.fori_loop(0, ROWS_PER_W, row_body, 0)


def _sc_combine(sims_flat, bidx, bval, vals_pad):
    mesh = plsc.VectorSubcoreMesh(core_axis_name="c", subcore_axis_name="s")
    fn = functools.partial(
        pl.kernel, mesh=mesh,
        compiler_params=pltpu.CompilerParams(needs_layout_passes=False),
        out_type=jax.ShapeDtypeStruct((N_OBJ, NUM_REL_PAD), jnp.float32),
        scratch_types=[
            pltpu.VMEM((ROWS_PER_W, TOPK), jnp.int32),
            pltpu.VMEM((ROWS_PER_W, TOPK), jnp.float32),
            pltpu.VMEM((TOPK,), jnp.int32),
            pltpu.VMEM((TOPK, BLK), jnp.float32),
            pltpu.VMEM((TOPK,), jnp.float32),
            pltpu.VMEM((TOPK,), jnp.float32),
            pltpu.VMEM((TOPK,), jnp.int32),
            pltpu.VMEM((TOPK, VAL_PAD), jnp.float32),
            pltpu.VMEM((NUM_REL_PAD,), jnp.float32),
            pltpu.SemaphoreType.DMA,
        ],
    )(_sc_body)
    return fn(sims_flat, bidx, bval, vals_pad)


def kernel(obj_logits, rel_pair_idxs, obj_embed, kg_keys, kg_vals):
    logits_pad = jnp.pad(obj_logits, ((0, 0), (0, NUM_OBJ_PAD - NUM_OBJ)),
                         constant_values=NEG)
    embed_pad = jnp.pad(obj_embed, ((0, NUM_OBJ_PAD - NUM_OBJ), (0, 0)))
    idx = rel_pair_idxs.astype(jnp.int32)
    keys_n = kg_keys / (jnp.linalg.norm(kg_keys, axis=-1, keepdims=True) + 1e-6)
    keys_pad = jnp.pad(keys_n, ((0, KG_PAD - KG), (0, 0)))
    vals_pad = jnp.pad(kg_vals, ((0, KG_PAD - KG), (0, VAL_PAD - NUM_REL)))

    qn = _prelude(logits_pad, embed_pad, idx)
    sims, m3 = _sims_and_blockmax(qn, keys_pad)
    m = m3.transpose(1, 0, 2).reshape(N_OBJ, NB)
    m_pad = jnp.pad(m, ((0, 0), (0, NB_PAD - NB)), constant_values=-3e38)
    bidx, bval = _select_blocks(m_pad)
    out64 = _sc_combine(sims.reshape(N_OBJ * NB, BLK), bidx, bval, vals_pad)
    return out64[:, :NUM_REL]


# speedup vs baseline: 7.6309x; 1.0241x over previous
import functools

import jax
import jax.numpy as jnp
from jax import lax
from jax.experimental import pallas as pl
from jax.experimental.pallas import tpu as pltpu
from jax.experimental.pallas import tpu_sc as plsc

N_OBJ = 1024
NUM_OBJ = 151
NUM_OBJ_PAD = 256
NUM_REL = 51
NUM_REL_PAD = 64
VAL_PAD = 128
D = 128
KG = 100000
BLK = 128
NB = 784
KG_PAD = NB * BLK
NB_PAD = 896
KB = 512
GSUB = KB // BLK
NSTEP = KG_PAD // KB
TOPK = 32
NEG = -1e30


def _prelude_body(l_ref, e_ref, i_ref, q_ref):
    lv = l_ref[...]
    m = jnp.max(lv, axis=1, keepdims=True)
    ex = jnp.exp(lv - m)
    p = ex / jnp.sum(ex, axis=1, keepdims=True)
    f = jnp.dot(p, e_ref[...], preferred_element_type=jnp.float32)
    ii = i_ref[...]
    col = lax.broadcasted_iota(jnp.int32, (N_OBJ, N_OBJ), 1)
    oh0 = jnp.where(col == ii[:, 0:1], 1.0, 0.0)
    oh1 = jnp.where(col == ii[:, 1:2], 1.0, 0.0)
    h = jnp.dot(oh0, f, preferred_element_type=jnp.float32,
                precision=lax.Precision.HIGHEST)
    t = jnp.dot(oh1, f, preferred_element_type=jnp.float32,
                precision=lax.Precision.HIGHEST)
    q = h * t
    nrm = jnp.sqrt(jnp.sum(q * q, axis=1, keepdims=True))
    q_ref[...] = q / (nrm + 1e-6)


def _prelude(logits_pad, embed_pad, idx):
    return pl.pallas_call(
        _prelude_body,
        out_shape=jax.ShapeDtypeStruct((N_OBJ, D), jnp.float32),
    )(logits_pad, embed_pad, idx)


def _sims_body(q_ref, k_ref, s_ref, m_ref):
    j = pl.program_id(0)
    s = lax.dot_general(q_ref[...], k_ref[...], (((1,), (1,)), ((), ())),
                        preferred_element_type=jnp.float32)
    cg = j * KB + lax.broadcasted_iota(jnp.int32, (1, KB), 1)
    s = jnp.where(cg < KG, s, NEG)
    s_ref[...] = s
    parts = [jnp.max(s[:, g * BLK:(g + 1) * BLK], axis=1, keepdims=True)
             for g in range(GSUB)]
    m_ref[0] = jnp.concatenate(parts, axis=1)


def _sims_and_blockmax(qn, keys_pad):
    return pl.pallas_call(
        _sims_body,
        grid=(NSTEP,),
        in_specs=[
            pl.BlockSpec((N_OBJ, D), lambda j: (0, 0)),
            pl.BlockSpec((KB, D), lambda j: (j, 0)),
        ],
        out_specs=[
            pl.BlockSpec((N_OBJ, KB), lambda j: (0, j)),
            pl.BlockSpec((1, N_OBJ, GSUB), lambda j: (j, 0, 0)),
        ],
        out_shape=[
            jax.ShapeDtypeStruct((N_OBJ, KG_PAD), jnp.float32),
            jax.ShapeDtypeStruct((NSTEP, N_OBJ, GSUB), jnp.float32),
        ],
    )(qn, keys_pad)


def _select_body(m_ref, bi_ref, bv_ref):
    a0 = m_ref[...]
    col = lax.broadcasted_iota(jnp.int32, (N_OBJ, NB_PAD), 1)
    k32 = lax.broadcasted_iota(jnp.int32, (N_OBJ, TOPK), 1)

    def body(k, carry):
        a, bi, bv = carry
        m = jnp.max(a, axis=1)
        am = jnp.min(jnp.where(a == m[:, None], col, 2**30), axis=1)
        bi = jnp.where(k32 == k, am[:, None], bi)
        bv = jnp.where(k32 == k, m[:, None], bv)
        a = jnp.where(col == am[:, None], -3e38, a)
        return (a, bi, bv)

    _, bi, bv = lax.fori_loop(
        0, TOPK, body,
        (a0, jnp.zeros((N_OBJ, TOPK), jnp.int32),
         jnp.zeros((N_OBJ, TOPK), jnp.float32)))
    bi_ref[...] = bi
    bv_ref[...] = bv


def _select_blocks(m_pad):
    return pl.pallas_call(
        _select_body,
        out_shape=[
            jax.ShapeDtypeStruct((N_OBJ, TOPK), jnp.int32),
            jax.ShapeDtypeStruct((N_OBJ, TOPK), jnp.float32),
        ],
    )(m_pad)


ROWS_PER_W = 32


def _splat_i(x):
    return jnp.full((16,), x, jnp.int32)


def _sc_body(sims_ref, bi_ref, bv_ref, vals_ref, out_ref,
             blk_v, sel_v, gid0, gid1, simsb0, simsb1, selrow, topv, topi,
             valsv, outrow, sem0, sem1, semv):
    wid = lax.axis_index("s") * 2 + lax.axis_index("c")
    base = wid * ROWS_PER_W
    pltpu.sync_copy(bi_ref.at[pl.ds(base, ROWS_PER_W)], blk_v)
    pltpu.sync_copy(bv_ref.at[pl.ds(base, ROWS_PER_W)], sel_v)
    lane = lax.broadcasted_iota(jnp.int32, (16,), 0)

    def start_gather(r, gid_v, simsb, sem):
        rsp = _splat_i(r)
        for c in range(2):
            b = plsc.load_gather(blk_v, [rsp, lane + (c * 16)])
            gid_v[pl.ds(c * 16, 16)] = b + (base + r) * NB
        pltpu.async_copy(sims_ref.at[gid_v], simsb, sem)

    start_gather(0, gid0, simsb0, sem0)
    start_gather(1, gid1, simsb1, sem1)

    def process_row(r, gid_v, simsb, sem):
        row = base + r
        rsp = _splat_i(r)
        pltpu.make_async_copy(sims_ref.at[gid_v], simsb, sem).wait()
        for c in range(2):
            selrow[pl.ds(c * 16, 16)] = plsc.load_gather(
                sel_v, [rsp, lane + (c * 16)])

        def sel_body(k, _):
            b0 = selrow[pl.ds(0, 16)]
            b1 = selrow[pl.ds(16, 16)]
            m = jnp.max(jnp.maximum(b0, b1))
            c0 = jnp.min(plsc.all_reduce_ffs(b0 == m))
            c1 = jnp.min(plsc.all_reduce_ffs(b1 == m))
            sstar = jnp.where(c0 < 16, c0, c1 + 16)
            ssp = _splat_i(sstar)
            pos = jnp.int32(1 << 20)
            for c in range(8):
                v = plsc.load_gather(simsb, [ssp, lane + (c * 16)])
                f = jnp.min(plsc.all_reduce_ffs(v == m))
                pos = jnp.minimum(pos,
                                  jnp.where(f < 16, c * 16 + f, 1 << 20))
            mask0 = lane == 0
            plsc.store_scatter(topv, [_splat_i(k)],
                               jnp.full((16,), m, jnp.float32), mask=mask0)
            blkid = jnp.max(plsc.load_gather(blk_v, [rsp, ssp]))
            plsc.store_scatter(topi, [_splat_i(k)],
                               _splat_i(blkid * BLK + pos), mask=mask0)
            plsc.store_scatter(simsb, [ssp, _splat_i(pos)],
                               jnp.full((16,), NEG, jnp.float32), mask=mask0)
            nm = jnp.full((16,), -3e38, jnp.float32)
            for c in range(8):
                nm = jnp.maximum(
                    nm, plsc.load_gather(simsb, [ssp, lane + (c * 16)]))
            plsc.store_scatter(selrow, [ssp],
                               jnp.full((16,), jnp.max(nm), jnp.float32),
                               mask=mask0)
            return 0

        lax.fori_loop(0, TOPK, sel_body, 0)

        t0 = topv[pl.ds(0, 16)]
        t1 = topv[pl.ds(16, 16)]
        m1 = jnp.max(jnp.maximum(t0, t1))
        w0 = jnp.exp((t0 - m1) * 10.0)
        w1 = jnp.exp((t1 - m1) * 10.0)
        ssum = jnp.full((16,), jnp.sum(w0) + jnp.sum(w1), jnp.float32)
        topv[pl.ds(0, 16)] = w0 / ssum
        topv[pl.ds(16, 16)] = w1 / ssum

        pltpu.async_copy(vals_ref.at[topi], valsv, semv).wait()
        for c in range(4):
            outrow[pl.ds(c * 16, 16)] = jnp.zeros((16,), jnp.float32)

        def acc_body(k, _):
            ksp = _splat_i(k)
            wk = jnp.max(plsc.load_gather(topv, [ksp]))
            for c in range(4):
                v = plsc.load_gather(valsv, [ksp, lane + (c * 16)])
                outrow[pl.ds(c * 16, 16)] = outrow[pl.ds(c * 16, 16)] + v * wk
            return 0

        lax.fori_loop(0, TOPK, acc_body, 0)
        pltpu.sync_copy(outrow, out_ref.at[row])

    def pair_body(i, _):
        r0 = 2 * i
        process_row(r0, gid0, simsb0, sem0)

        @pl.when(r0 + 2 < ROWS_PER_W)
        def _():
            start_gather(r0 + 2, gid0, simsb0, sem0)

        process_row(r0 + 1, gid1, simsb1, sem1)

        @pl.when(r0 + 3 < ROWS_PER_W)
        def _():
            start_gather(r0 + 3, gid1, simsb1, sem1)

        return 0

    lax.fori_loop(0, ROWS_PER_W // 2, pair_body, 0)


def _sc_combine(sims_flat, bidx, bval, vals_pad):
    mesh = plsc.VectorSubcoreMesh(core_axis_name="c", subcore_axis_name="s")
    fn = functools.partial(
        pl.kernel, mesh=mesh,
        compiler_params=pltpu.CompilerParams(needs_layout_passes=False),
        out_type=jax.ShapeDtypeStruct((N_OBJ, NUM_REL_PAD), jnp.float32),
        scratch_types=[
            pltpu.VMEM((ROWS_PER_W, TOPK), jnp.int32),
            pltpu.VMEM((ROWS_PER_W, TOPK), jnp.float32),
            pltpu.VMEM((TOPK,), jnp.int32),
            pltpu.VMEM((TOPK,), jnp.int32),
            pltpu.VMEM((TOPK, BLK), jnp.float32),
            pltpu.VMEM((TOPK, BLK), jnp.float32),
            pltpu.VMEM((TOPK,), jnp.float32),
            pltpu.VMEM((TOPK,), jnp.float32),
            pltpu.VMEM((TOPK,), jnp.int32),
            pltpu.VMEM((TOPK, VAL_PAD), jnp.float32),
            pltpu.VMEM((NUM_REL_PAD,), jnp.float32),
            pltpu.SemaphoreType.DMA,
            pltpu.SemaphoreType.DMA,
            pltpu.SemaphoreType.DMA,
        ],
    )(_sc_body)
    return fn(sims_flat, bidx, bval, vals_pad)


def kernel(obj_logits, rel_pair_idxs, obj_embed, kg_keys, kg_vals):
    logits_pad = jnp.pad(obj_logits, ((0, 0), (0, NUM_OBJ_PAD - NUM_OBJ)),
                         constant_values=NEG)
    embed_pad = jnp.pad(obj_embed, ((0, NUM_OBJ_PAD - NUM_OBJ), (0, 0)))
    idx = rel_pair_idxs.astype(jnp.int32)
    keys_n = kg_keys / (jnp.linalg.norm(kg_keys, axis=-1, keepdims=True) + 1e-6)
    keys_pad = jnp.pad(keys_n, ((0, KG_PAD - KG), (0, 0)))
    vals_pad = jnp.pad(kg_vals, ((0, KG_PAD - KG), (0, VAL_PAD - NUM_REL)))

    qn = _prelude(logits_pad, embed_pad, idx)
    sims, m3 = _sims_and_blockmax(qn, keys_pad)
    m = m3.transpose(1, 0, 2).reshape(N_OBJ, NB)
    m_pad = jnp.pad(m, ((0, 0), (0, NB_PAD - NB)), constant_values=-3e38)
    bidx, bval = _select_blocks(m_pad)
    out64 = _sc_combine(sims.reshape(N_OBJ * NB, BLK), bidx, bval, vals_pad)
    return out64[:, :NUM_REL]
